# Initial kernel scaffold; baseline (speedup 1.0000x reference)
import jax
import jax.numpy as jnp
from jax.experimental import pallas as pl


def kernel(x, support, W1, b1, W2, b2, Wp, bp):
    raise NotImplementedError("write your pallas kernel here")


# speedup vs baseline: 1.1880x; 1.1880x over previous
import jax
import jax.numpy as jnp
from jax.experimental import pallas as pl
from jax.experimental.pallas import tpu as pltpu


def _gcn_kernel(x_ref, s_ref, w1_ref, b1_ref, w2_ref, b2_ref, wp_ref,
                bp_ref, o_ref):
    x = x_ref[0]
    s = s_ref[0]
    t = jnp.dot(x, w1_ref[...], preferred_element_type=jnp.float32)
    h = jnp.dot(s, t, preferred_element_type=jnp.float32) + b1_ref[...]
    h = jnp.maximum(h, 0.0)
    t2 = jnp.dot(h, w2_ref[...], preferred_element_type=jnp.float32)
    h2 = jnp.dot(s, t2, preferred_element_type=jnp.float32) + b2_ref[...]
    h2 = jnp.maximum(h2, 0.0)
    mx = jnp.max(h2, axis=0, keepdims=True)
    sm = jnp.sum(h2, axis=0, keepdims=True)
    cat = jnp.concatenate([mx, sm], axis=1)
    o_ref[0] = jnp.dot(cat, wp_ref[...],
                       preferred_element_type=jnp.float32) + bp_ref[...]


def kernel(x, support, W1, b1, W2, b2, Wp, bp):
    B, N, D_IN = x.shape
    H1 = W1.shape[1]
    H2 = W2.shape[1]
    OUT = Wp.shape[1]

    b1_2d = b1.reshape(1, H1)
    b2_2d = b2.reshape(1, H2)
    bp_2d = bp.reshape(1, OUT)

    out = pl.pallas_call(
        _gcn_kernel,
        grid=(B,),
        in_specs=[
            pl.BlockSpec((1, N, D_IN), lambda b: (b, 0, 0)),
            pl.BlockSpec((1, N, N), lambda b: (b, 0, 0)),
            pl.BlockSpec((D_IN, H1), lambda b: (0, 0)),
            pl.BlockSpec((1, H1), lambda b: (0, 0)),
            pl.BlockSpec((H1, H2), lambda b: (0, 0)),
            pl.BlockSpec((1, H2), lambda b: (0, 0)),
            pl.BlockSpec((2 * H2, OUT), lambda b: (0, 0)),
            pl.BlockSpec((1, OUT), lambda b: (0, 0)),
        ],
        out_specs=pl.BlockSpec((1, 1, OUT), lambda b: (b, 0, 0)),
        out_shape=jax.ShapeDtypeStruct((B, 1, OUT), jnp.float32),
        compiler_params=pltpu.CompilerParams(
            vmem_limit_bytes=100 * 1024 * 1024,
        ),
    )(x, support, W1, b1_2d, W2, b2_2d, Wp, bp_2d)
    return out.reshape(B, OUT)


# speedup vs baseline: 1.2079x; 1.0167x over previous
import jax
import jax.numpy as jnp
from jax.experimental import pallas as pl
from jax.experimental.pallas import tpu as pltpu


def _gcn_kernel(x_ref, s_ref, w1_ref, b1_ref, w2_ref, b2_ref, wp_ref,
                bp_ref, o_ref):
    x = x_ref[0]
    s = s_ref[0]
    t = jnp.dot(x, w1_ref[...], preferred_element_type=jnp.float32)
    h = jnp.dot(s, t, preferred_element_type=jnp.float32) + b1_ref[...]
    h = jnp.maximum(h, 0.0)
    t2 = jnp.dot(h, w2_ref[...], preferred_element_type=jnp.float32)
    h2 = jnp.dot(s, t2, preferred_element_type=jnp.float32) + b2_ref[...]
    h2 = jnp.maximum(h2, 0.0)
    mx = jnp.max(h2, axis=0, keepdims=True)
    sm = jnp.sum(h2, axis=0, keepdims=True)
    cat = jnp.concatenate([mx, sm], axis=1)
    o_ref[0] = jnp.dot(cat, wp_ref[...],
                       preferred_element_type=jnp.float32) + bp_ref[...]


def kernel(x, support, W1, b1, W2, b2, Wp, bp):
    B, N, D_IN = x.shape
    H1 = W1.shape[1]
    H2 = W2.shape[1]
    OUT = Wp.shape[1]

    b1_2d = b1.reshape(1, H1)
    b2_2d = b2.reshape(1, H2)
    bp_2d = bp.reshape(1, OUT)

    out = pl.pallas_call(
        _gcn_kernel,
        grid=(B,),
        in_specs=[
            pl.BlockSpec((1, N, D_IN), lambda b: (b, 0, 0)),
            pl.BlockSpec((1, N, N), lambda b: (b, 0, 0)),
            pl.BlockSpec((D_IN, H1), lambda b: (0, 0)),
            pl.BlockSpec((1, H1), lambda b: (0, 0)),
            pl.BlockSpec((H1, H2), lambda b: (0, 0)),
            pl.BlockSpec((1, H2), lambda b: (0, 0)),
            pl.BlockSpec((2 * H2, OUT), lambda b: (0, 0)),
            pl.BlockSpec((1, OUT), lambda b: (0, 0)),
        ],
        out_specs=pl.BlockSpec((1, 1, OUT), lambda b: (b, 0, 0)),
        out_shape=jax.ShapeDtypeStruct((B, 1, OUT), jnp.float32),
        compiler_params=pltpu.CompilerParams(
            vmem_limit_bytes=100 * 1024 * 1024,
            dimension_semantics=("parallel",),
        ),
    )(x, support, W1, b1_2d, W2, b2_2d, Wp, bp_2d)
    return out.reshape(B, OUT)
